# baseline (speedup 1.0000x reference)
import jax
import jax.numpy as jnp
from jax.experimental import pallas as pl


_N = 512
_K = 1024
_D = 256


def _vq_kernel(z_ref, w_ref, idx_ref, zq_ref, ma_ref):
    w = w_ref[...]

    def body(i, _):
        zrow = z_ref[pl.ds(i, 1), :]
        diff = zrow - w
        sq = diff * diff
        d = jnp.sum(sq, axis=1, keepdims=True)
        dmin = jnp.min(d)
        iota = jax.lax.broadcasted_iota(jnp.int32, (_K, 1), 0)
        cand = jnp.where(d == dmin, iota, _K)
        idx = jnp.min(cand)
        idx_ref[pl.ds(i, 1), :] = jnp.full((1, 1), idx, jnp.int32)
        zqrow = w_ref[pl.ds(idx, 1), :]
        zq_ref[pl.ds(i, 1), :] = zqrow
        ma_ref[pl.ds(i, 1), :] = zrow + (zqrow - zrow)
        return 0

    jax.lax.fori_loop(0, _N, body, 0)


def kernel(x, weight):
    z = jnp.transpose(x, (0, 2, 3, 1))
    zf = z.reshape(_N, _D)
    idx2, zqf, maf = pl.pallas_call(
        _vq_kernel,
        out_shape=(
            jax.ShapeDtypeStruct((_N, 1), jnp.int32),
            jax.ShapeDtypeStruct((_N, _D), jnp.float32),
            jax.ShapeDtypeStruct((_N, _D), jnp.float32),
        ),
    )(zf, weight)
    indices = idx2.reshape(_N)
    z_q = zqf.reshape(z.shape)
    z_q_ma = jnp.transpose(maf.reshape(z.shape), (0, 3, 1, 2))
    return (z_q_ma, z_q, z, indices)
